# baseline (speedup 1.0000x reference)
import functools
import math

import jax
import jax.numpy as jnp
from jax import lax
from jax.experimental import pallas as pl

GROUP = 16
SUB = 2


def _dotT(a, b):
    return lax.dot_general(a, b, (((1,), (1,)), ((), ())),
                           preferred_element_type=jnp.float32)


def _dot(a, b):
    return jnp.dot(a, b, preferred_element_type=jnp.float32)


def _seg_body(SS, EPS, HEADS, GROUP, SUB,
              nf_ref, df_ref, nn_ref, nd_ref, ds_ref,
              fwWq_ref, fwWk_ref, fwWv_ref, fwWo_ref,
              fm0_ref, fm1_ref, fm2_ref,
              rvWq_ref, rvWk_ref, rvWv_ref, rvWo_ref,
              rm0_ref, rm1_ref, rm2_ref,
              nout_ref, dout_ref):
    s = pl.program_id(0)
    WB = SUB * SS
    EB = SUB * EPS
    HS = HEADS * WB
    pos = jnp.bitwise_and(
        lax.broadcasted_iota(jnp.int32, (WB, 1), 0), SS - 1).astype(
        jnp.float32) * jnp.float32(1.0 / 64.0)
    shift = WB.bit_length() - 1
    bi = lax.broadcasted_iota(jnp.int32, (HS, HEADS), 0)
    bh = lax.broadcasted_iota(jnp.int32, (HS, HEADS), 1)
    Bsum = (lax.shift_right_logical(bi, shift) == bh).astype(jnp.float32)
    iota = lax.broadcasted_iota(jnp.int32, (EB, WB), 1)
    iota_t = jnp.bitwise_and(
        lax.broadcasted_iota(jnp.int32, (EB, HS), 1), WB - 1)

    SIZE = nf_ref.shape[1]
    DS = df_ref.shape[1]
    A_fw = fwWq_ref.shape[1] // HEADS
    A_rv = rvWq_ref.shape[1] // HEADS
    inv_sqrt_a = jnp.float32(1.0 / math.sqrt(A_fw))
    inv_sqrt_rv = jnp.float32(1.0 / math.sqrt(A_rv))

    for g in range(GROUP // SUB):
        nf = nf_ref[g * WB:(g + 1) * WB, :]
        df = df_ref[g * EB:(g + 1) * EB, :]
        nd = nd_ref[g * EB:(g + 1) * EB, :]
        nn = nn_ref[g * WB:(g + 1) * WB, :]
        ds = ds_ref[g * EB:(g + 1) * EB, :]

        a_loc = ds[:, 0:1] - (s * GROUP * SS + g * WB)
        b_loc = ds[:, 1:2] - (s * GROUP * SS + g * WB)
        Ma = (iota == a_loc).astype(jnp.float32)
        Mb = (iota == b_loc).astype(jnp.float32)
        cmp_a = iota_t == a_loc
        cmp_b = iota_t == b_loc

        k_fw = _dot(nf, fwWk_ref[0:SIZE, :]) + pos * fwWk_ref[SIZE:SIZE + 1, :]
        v_fw = _dot(nf, fwWv_ref[0:SIZE, :]) + pos * fwWv_ref[SIZE:SIZE + 1, :]
        dfn = df + 0.1 * nd

        K2 = jnp.concatenate(
            [_dotT(fwWq_ref[:, h * A_fw:(h + 1) * A_fw],
                   k_fw[:, h * A_fw:(h + 1) * A_fw]) for h in range(HEADS)],
            axis=1)
        L = _dot(dfn * inv_sqrt_a, K2)
        zero = jnp.float32(0.0)
        S = jnp.where(cmp_a, L, zero) - jnp.where(cmp_b, L, zero)
        dl = _dot(S, Bsum)
        wa = 1.0 / (1.0 + jnp.exp(-dl))
        wa_t = _dotT(wa, Bsum)
        P = jnp.where(cmp_a, wa_t,
                      jnp.where(cmp_b, 1.0 - wa_t, zero))
        vo = jnp.concatenate(
            [_dot(v_fw[:, h * A_fw:(h + 1) * A_fw],
                  fwWo_ref[h * A_fw:(h + 1) * A_fw, :]) for h in range(HEADS)],
            axis=0)
        d1 = dfn + _dot(P, vo)
        h0 = jnp.maximum(_dot(d1, fm0_ref[:]), 0.0)
        h1 = jnp.maximum(_dot(h0, fm1_ref[:]), 0.0)
        dout_ref[g * EB:(g + 1) * EB, :] = d1 + _dot(h1, fm2_ref[:])

        dp0 = _dot(Mb, pos) * jnp.float32(1.0 / 64.0)
        dp1 = _dot(Ma, pos) * jnp.float32(1.0 / 64.0)
        k_rv = (_dot(df, rvWk_ref[0:DS, :]) + dp0 * rvWk_ref[DS:DS + 1, :]
                + dp1 * rvWk_ref[DS + 1:DS + 2, :])
        v_rv = (_dot(df, rvWv_ref[0:DS, :]) + dp0 * rvWv_ref[DS:DS + 1, :]
                + dp1 * rvWv_ref[DS + 1:DS + 2, :])
        nfn = nf + 0.1 * nn
        q_rv = _dot(nfn, rvWq_ref[:])

        q_rv = q_rv * inv_sqrt_rv
        Lr = jnp.concatenate(
            [_dotT(q_rv[:, h * A_rv:(h + 1) * A_rv],
                   k_rv[:, h * A_rv:(h + 1) * A_rv]) for h in range(HEADS)],
            axis=0)
        valid_rv = jnp.transpose(Ma + Mb)
        valid_t = jnp.concatenate([valid_rv] * HEADS, axis=0)
        m = jnp.max(Lr, axis=-1, keepdims=True)
        e = jnp.exp(Lr - m) * valid_t
        Pr = e / jnp.sum(e, axis=-1, keepdims=True)
        n_attn = jnp.concatenate(
            [_dot(Pr[h * WB:(h + 1) * WB, :],
                  v_rv[:, h * A_rv:(h + 1) * A_rv]) for h in range(HEADS)],
            axis=1)
        n1 = nfn + _dot(n_attn, rvWo_ref[:])
        g0 = jnp.maximum(_dot(n1, rm0_ref[:]), 0.0)
        g1 = jnp.maximum(_dot(g0, rm1_ref[:]), 0.0)
        nout_ref[g * WB:(g + 1) * WB, :] = n1 + _dot(g1, rm2_ref[:])


def kernel(node_features, distance_features, node_structure,
           distance_structure, subgraph_indices, noise_node, noise_dist,
           fw_Wq, fw_Wk, fw_Wv, fw_Wo, fw_m0, fw_m1, fw_m2,
           rv_Wq, rv_Wk, rv_Wv, rv_Wo, rv_m0, rv_m1, rv_m2):
    N, SIZE = node_features.shape
    E, DSIZE = distance_features.shape
    SEG_SIZE = node_structure.shape[1] + 1
    SEG = N // SEG_SIZE
    EPS = E // SEG
    HEADS = 8
    W = GROUP * SEG_SIZE
    EPW = GROUP * EPS

    ds32 = distance_structure.astype(jnp.int32)

    f32 = jnp.float32
    out_shape = (jax.ShapeDtypeStruct((N, SIZE), f32),
                 jax.ShapeDtypeStruct((E, DSIZE), f32))

    seg_map = lambda s: (s, 0)
    fix_map = lambda s: (0, 0)

    def w_spec(w):
        return pl.BlockSpec(w.shape, fix_map)

    grid_specs = dict(
        grid=(SEG // GROUP,),
        in_specs=[
            pl.BlockSpec((W, SIZE), seg_map),
            pl.BlockSpec((EPW, DSIZE), seg_map),
            pl.BlockSpec((W, SIZE), seg_map),
            pl.BlockSpec((EPW, DSIZE), seg_map),
            pl.BlockSpec((EPW, 2), seg_map),
            w_spec(fw_Wq), w_spec(fw_Wk), w_spec(fw_Wv),
            w_spec(fw_Wo), w_spec(fw_m0), w_spec(fw_m1), w_spec(fw_m2),
            w_spec(rv_Wq), w_spec(rv_Wk), w_spec(rv_Wv),
            w_spec(rv_Wo), w_spec(rv_m0), w_spec(rv_m1), w_spec(rv_m2),
        ],
        out_specs=[
            pl.BlockSpec((W, SIZE), seg_map),
            pl.BlockSpec((EPW, DSIZE), seg_map),
        ],
    )

    body = functools.partial(_seg_body, SEG_SIZE, EPS, HEADS, GROUP, SUB)
    n_out, d_out = pl.pallas_call(
        body,
        out_shape=out_shape,
        **grid_specs,
    )(node_features, distance_features, noise_node, noise_dist, ds32,
      fw_Wq, fw_Wk, fw_Wv, fw_Wo, fw_m0, fw_m1, fw_m2,
      rv_Wq, rv_Wk, rv_Wv, rv_Wo, rv_m0, rv_m1, rv_m2)
    return (n_out, d_out)


# speedup vs baseline: 1.2450x; 1.2450x over previous
import functools
import math

import jax
import jax.numpy as jnp
from jax import lax
from jax.experimental import pallas as pl

GROUP = 8
SUB = 2


def _dotT(a, b):
    return lax.dot_general(a, b, (((1,), (1,)), ((), ())),
                           preferred_element_type=jnp.float32)


def _dot(a, b):
    return jnp.dot(a, b, preferred_element_type=jnp.float32)


def _seg_body(SS, EPS, HEADS, GROUP, SUB,
              nf_ref, df_ref, nn_ref, nd_ref, ds_ref,
              fwWq_ref, fwWk_ref, fwWv_ref, fwWo_ref,
              fm0_ref, fm1_ref, fm2_ref,
              rvWq_ref, rvWk_ref, rvWv_ref, rvWo_ref,
              rm0_ref, rm1_ref, rm2_ref,
              nout_ref, dout_ref):
    s = pl.program_id(0)
    WB = SUB * SS
    EB = SUB * EPS
    HS = HEADS * WB
    pos = jnp.bitwise_and(
        lax.broadcasted_iota(jnp.int32, (WB, 1), 0), SS - 1).astype(
        jnp.float32) * jnp.float32(1.0 / 64.0)
    shift = WB.bit_length() - 1
    bi = lax.broadcasted_iota(jnp.int32, (HS, HEADS), 0)
    bh = lax.broadcasted_iota(jnp.int32, (HS, HEADS), 1)
    Bsum = (lax.shift_right_logical(bi, shift) == bh).astype(jnp.float32)
    iota = lax.broadcasted_iota(jnp.int32, (EB, WB), 1)
    iota_t = jnp.bitwise_and(
        lax.broadcasted_iota(jnp.int32, (EB, HS), 1), WB - 1)

    SIZE = nf_ref.shape[1]
    DS = df_ref.shape[1]
    A_fw = fwWq_ref.shape[1] // HEADS
    A_rv = rvWq_ref.shape[1] // HEADS
    inv_sqrt_a = jnp.float32(1.0 / math.sqrt(A_fw))
    inv_sqrt_rv = jnp.float32(1.0 / math.sqrt(A_rv))

    for g in range(GROUP // SUB):
        nf = nf_ref[g * WB:(g + 1) * WB, :]
        df = df_ref[g * EB:(g + 1) * EB, :]
        nd = nd_ref[g * EB:(g + 1) * EB, :]
        nn = nn_ref[g * WB:(g + 1) * WB, :]
        ds = ds_ref[g * EB:(g + 1) * EB, :]

        a_loc = ds[:, 0:1] - (s * GROUP * SS + g * WB)
        b_loc = ds[:, 1:2] - (s * GROUP * SS + g * WB)
        Ma = (iota == a_loc).astype(jnp.float32)
        Mb = (iota == b_loc).astype(jnp.float32)
        cmp_a = iota_t == a_loc
        cmp_b = iota_t == b_loc

        k_fw = _dot(nf, fwWk_ref[0:SIZE, :]) + pos * fwWk_ref[SIZE:SIZE + 1, :]
        v_fw = _dot(nf, fwWv_ref[0:SIZE, :]) + pos * fwWv_ref[SIZE:SIZE + 1, :]
        dfn = df + 0.1 * nd

        K2 = jnp.concatenate(
            [_dotT(fwWq_ref[:, h * A_fw:(h + 1) * A_fw],
                   k_fw[:, h * A_fw:(h + 1) * A_fw]) for h in range(HEADS)],
            axis=1)
        L = _dot(dfn * inv_sqrt_a, K2)
        zero = jnp.float32(0.0)
        S = jnp.where(cmp_a, L, zero) - jnp.where(cmp_b, L, zero)
        dl = _dot(S, Bsum)
        wa = 1.0 / (1.0 + jnp.exp(-dl))
        wa_t = _dotT(wa, Bsum)
        P = jnp.where(cmp_a, wa_t,
                      jnp.where(cmp_b, 1.0 - wa_t, zero))
        vo = jnp.concatenate(
            [_dot(v_fw[:, h * A_fw:(h + 1) * A_fw],
                  fwWo_ref[h * A_fw:(h + 1) * A_fw, :]) for h in range(HEADS)],
            axis=0)
        d1 = dfn + _dot(P, vo)
        h0 = jnp.maximum(_dot(d1, fm0_ref[:]), 0.0)
        h1 = jnp.maximum(_dot(h0, fm1_ref[:]), 0.0)
        dout_ref[g * EB:(g + 1) * EB, :] = d1 + _dot(h1, fm2_ref[:])

        dp0 = _dot(Mb, pos) * jnp.float32(1.0 / 64.0)
        dp1 = _dot(Ma, pos) * jnp.float32(1.0 / 64.0)
        k_rv = (_dot(df, rvWk_ref[0:DS, :]) + dp0 * rvWk_ref[DS:DS + 1, :]
                + dp1 * rvWk_ref[DS + 1:DS + 2, :])
        v_rv = (_dot(df, rvWv_ref[0:DS, :]) + dp0 * rvWv_ref[DS:DS + 1, :]
                + dp1 * rvWv_ref[DS + 1:DS + 2, :])
        nfn = nf + 0.1 * nn
        q_rv = _dot(nfn, rvWq_ref[:])

        q_rv = q_rv * inv_sqrt_rv
        Lr = jnp.concatenate(
            [_dotT(q_rv[:, h * A_rv:(h + 1) * A_rv],
                   k_rv[:, h * A_rv:(h + 1) * A_rv]) for h in range(HEADS)],
            axis=0)
        valid_rv = jnp.transpose(Ma + Mb)
        valid_t = jnp.concatenate([valid_rv] * HEADS, axis=0)
        m = jnp.max(Lr, axis=-1, keepdims=True)
        e = jnp.exp(Lr - m) * valid_t
        Pr = e / jnp.sum(e, axis=-1, keepdims=True)
        n_attn = jnp.concatenate(
            [_dot(Pr[h * WB:(h + 1) * WB, :],
                  v_rv[:, h * A_rv:(h + 1) * A_rv]) for h in range(HEADS)],
            axis=1)
        n1 = nfn + _dot(n_attn, rvWo_ref[:])
        g0 = jnp.maximum(_dot(n1, rm0_ref[:]), 0.0)
        g1 = jnp.maximum(_dot(g0, rm1_ref[:]), 0.0)
        nout_ref[g * WB:(g + 1) * WB, :] = n1 + _dot(g1, rm2_ref[:])


def kernel(node_features, distance_features, node_structure,
           distance_structure, subgraph_indices, noise_node, noise_dist,
           fw_Wq, fw_Wk, fw_Wv, fw_Wo, fw_m0, fw_m1, fw_m2,
           rv_Wq, rv_Wk, rv_Wv, rv_Wo, rv_m0, rv_m1, rv_m2):
    N, SIZE = node_features.shape
    E, DSIZE = distance_features.shape
    SEG_SIZE = node_structure.shape[1] + 1
    SEG = N // SEG_SIZE
    EPS = E // SEG
    HEADS = 8
    W = GROUP * SEG_SIZE
    EPW = GROUP * EPS

    ds32 = distance_structure.astype(jnp.int32)

    f32 = jnp.float32
    out_shape = (jax.ShapeDtypeStruct((N, SIZE), f32),
                 jax.ShapeDtypeStruct((E, DSIZE), f32))

    seg_map = lambda s: (s, 0)
    fix_map = lambda s: (0, 0)

    def w_spec(w):
        return pl.BlockSpec(w.shape, fix_map)

    grid_specs = dict(
        grid=(SEG // GROUP,),
        in_specs=[
            pl.BlockSpec((W, SIZE), seg_map),
            pl.BlockSpec((EPW, DSIZE), seg_map),
            pl.BlockSpec((W, SIZE), seg_map),
            pl.BlockSpec((EPW, DSIZE), seg_map),
            pl.BlockSpec((EPW, 2), seg_map),
            w_spec(fw_Wq), w_spec(fw_Wk), w_spec(fw_Wv),
            w_spec(fw_Wo), w_spec(fw_m0), w_spec(fw_m1), w_spec(fw_m2),
            w_spec(rv_Wq), w_spec(rv_Wk), w_spec(rv_Wv),
            w_spec(rv_Wo), w_spec(rv_m0), w_spec(rv_m1), w_spec(rv_m2),
        ],
        out_specs=[
            pl.BlockSpec((W, SIZE), seg_map),
            pl.BlockSpec((EPW, DSIZE), seg_map),
        ],
    )

    body = functools.partial(_seg_body, SEG_SIZE, EPS, HEADS, GROUP, SUB)
    n_out, d_out = pl.pallas_call(
        body,
        out_shape=out_shape,
        **grid_specs,
    )(node_features, distance_features, noise_node, noise_dist, ds32,
      fw_Wq, fw_Wk, fw_Wv, fw_Wo, fw_m0, fw_m1, fw_m2,
      rv_Wq, rv_Wk, rv_Wv, rv_Wo, rv_m0, rv_m1, rv_m2)
    return (n_out, d_out)
